# Initial kernel scaffold; baseline (speedup 1.0000x reference)
import jax
import jax.numpy as jnp
from jax.experimental import pallas as pl


def kernel(w_p, w_l, w_u, s_sign, w_s, l_mask, u_mask, w0, b0, w2, b2, wz, bz, scale_z, x):
    raise NotImplementedError("write your pallas kernel here")


# speedup vs baseline: 2.7211x; 2.7211x over previous
import jax
import jax.numpy as jnp
from jax.experimental import pallas as pl
from jax.experimental.pallas import tpu as pltpu

_NB = 4
_NS = 64


def _rot_lanes(arr, k, size):
    k = k % size
    if k == 0:
        return arr
    return jnp.concatenate([arr[:, k:], arr[:, :k]], axis=1)


def _stats_kernel(x_ref, out_ref):
    x = x_ref[...]
    s = jnp.sum(x, axis=(0, 2), keepdims=True)
    q = jnp.sum(x * x, axis=(0, 2), keepdims=True)
    out_ref[...] = jnp.concatenate([s, q], axis=2)


def _make_main_kernel(nb, C, C2, F, H, W):
    HW = H * W
    offs = [(dy, dx) for dy in (-1, 0, 1) for dx in (-1, 0, 1)]

    def kern(x_ref, ms_ref, wc_ref, w0_ref, w2_ref, wz_ref, b01_ref, rs_ref,
             act_ref, w_out_ref, out_ref, det_ref):
        col = jax.lax.broadcasted_iota(jnp.int32, (1, HW), 1)
        xpos = col % W
        ypos = col // W
        oks = [((xpos + dx >= 0) & (xpos + dx < W) &
                (ypos + dy >= 0) & (ypos + dy < H)) for dy, dx in offs]

        mean = ms_ref[:, 0:1]
        scale = ms_ref[:, 1:2]
        wc = wc_ref[...]
        w0 = w0_ref[...]
        w2 = w2_ref[...]
        b0 = b01_ref[:, 0:1]
        b2 = b01_ref[:, 1:2]

        dets = []
        for i in range(nb):
            xs = x_ref[i]
            a = scale * (xs - mean)
            act_ref[i] = a
            w = jnp.dot(wc, a, preferred_element_type=jnp.float32)
            w_out_ref[i] = w
            in_a = w[0:C2]
            in_b = w[C2:C]

            slabs = [jnp.where(oks[t], _rot_lanes(in_a, dy * W + dx, HW), 0.0)
                     for t, (dy, dx) in enumerate(offs)]
            patch = jnp.concatenate(slabs, axis=0).astype(jnp.bfloat16)
            h1 = jnp.dot(w0, patch, preferred_element_type=jnp.float32)
            h1 = jnp.maximum(h1.astype(jnp.bfloat16) + b0, 0)

            h2 = jnp.dot(w2, h1, preferred_element_type=jnp.float32)
            h2 = jnp.maximum(h2.astype(jnp.bfloat16) + b2, 0)

            acc = None
            for t, (dy, dx) in enumerate(offs):
                g = jnp.dot(wz_ref[t], h2, preferred_element_type=jnp.float32)
                contrib = jnp.where(oks[t], _rot_lanes(g, dy * W + dx, HW),
                                    rs_ref[:, t:t + 1])
                acc = contrib if acc is None else acc + contrib
            net = acc + rs_ref[:, 9:10]

            s = jax.nn.sigmoid(net[0:C2] + 2.0)
            tt = net[C2:C]
            out_ref[i] = jnp.concatenate([w[0:C2], (in_b + tt) * s], axis=0)
            dets.append(jnp.sum(jnp.log(s)))

        rows = [jnp.zeros((1, 128), jnp.float32) + d for d in dets]
        if nb < 8:
            rows.append(jnp.zeros((8 - nb, 128), jnp.float32))
        det_ref[0] = jnp.concatenate(rows, axis=0)

    return kern


def kernel(w_p, w_l, w_u, s_sign, w_s, l_mask, u_mask,
           w0, b0, w2, b2, wz, bz, scale_z, x):
    N, C, H, W = x.shape
    C2 = C // 2
    F = w0.shape[0]
    HW = H * W
    M = N * HW
    x3 = x.reshape(N, C, HW)

    ns = _NS if N % _NS == 0 else N
    nb = _NB if N % _NB == 0 else 1
    G = N // ns

    parts = pl.pallas_call(
        _stats_kernel,
        grid=(G,),
        in_specs=[pl.BlockSpec((ns, C, HW), lambda g: (g, 0, 0))],
        out_specs=pl.BlockSpec((1, C, 2), lambda g: (g, 0, 0)),
        out_shape=jax.ShapeDtypeStruct((G, C, 2), jnp.float32),
        compiler_params=pltpu.CompilerParams(dimension_semantics=("parallel",)),
    )(x3)
    stats = jnp.sum(parts, axis=0)

    mean = stats[:, 0] / M
    var = (stats[:, 1] - M * mean * mean) / (M - 1)
    scale = 1.0 / (jnp.sqrt(var) + 1e-6)
    logdet_act = float(HW) * jnp.sum(jnp.log(jnp.abs(scale)))
    det1 = float(HW) * jnp.sum(w_s)
    ms = jnp.stack([mean, scale], axis=1)

    l = w_l * l_mask + jnp.eye(C, dtype=jnp.float32)
    u = w_u * u_mask + jnp.diag(s_sign * jnp.exp(w_s))
    wc = w_p @ l @ u
    w0_2d = w0.transpose(0, 2, 3, 1).reshape(F, 9 * C2).astype(jnp.bfloat16)
    w2_2d = w2[:, :, 0, 0].astype(jnp.bfloat16)
    cs = jnp.exp(scale_z * 3.0)
    wz_t = (wz * cs[:, None, None, None]).transpose(2, 3, 0, 1).reshape(9, C, F)
    rs_mat = jnp.concatenate([jnp.sum(wz_t, axis=2).T, (bz * cs)[:, None]],
                             axis=1)
    wz_b = wz_t.astype(jnp.bfloat16)
    b01 = jnp.stack([b0, b2], axis=1).astype(jnp.bfloat16)

    act3, w3, out3, det_blk = pl.pallas_call(
        _make_main_kernel(nb, C, C2, F, H, W),
        grid=(N // nb,),
        in_specs=[pl.BlockSpec((nb, C, HW), lambda n: (n, 0, 0)),
                  pl.BlockSpec((C, 2), lambda n: (0, 0)),
                  pl.BlockSpec((C, C), lambda n: (0, 0)),
                  pl.BlockSpec((F, 9 * C2), lambda n: (0, 0)),
                  pl.BlockSpec((F, F), lambda n: (0, 0)),
                  pl.BlockSpec((9, C, F), lambda n: (0, 0, 0)),
                  pl.BlockSpec((F, 2), lambda n: (0, 0)),
                  pl.BlockSpec((C, 10), lambda n: (0, 0))],
        out_specs=[pl.BlockSpec((nb, C, HW), lambda n: (n, 0, 0)),
                   pl.BlockSpec((nb, C, HW), lambda n: (n, 0, 0)),
                   pl.BlockSpec((nb, C, HW), lambda n: (n, 0, 0)),
                   pl.BlockSpec((1, 8, 128), lambda n: (n, 0, 0))],
        out_shape=[jax.ShapeDtypeStruct((N, C, HW), jnp.float32),
                   jax.ShapeDtypeStruct((N, C, HW), jnp.float32),
                   jax.ShapeDtypeStruct((N, C, HW), jnp.float32),
                   jax.ShapeDtypeStruct((N // nb, 8, 128), jnp.float32)],
        compiler_params=pltpu.CompilerParams(dimension_semantics=("parallel",)),
    )(x3, ms, wc, w0_2d, w2_2d, wz_b, b01, rs_mat)

    det2 = det_blk[:, 0:nb, 0].reshape(N)
    logdet = logdet_act + det1 + det2
    return (act3.reshape(N, C, H, W), w3.reshape(N, C, H, W),
            out3.reshape(N, C, H, W), logdet)


# speedup vs baseline: 4.7889x; 1.7599x over previous
import jax
import jax.numpy as jnp
from jax.experimental import pallas as pl
from jax.experimental.pallas import tpu as pltpu

_NB = 4
_NS = 64


def _rot_lanes(arr, k, size):
    k = k % size
    if k == 0:
        return arr
    return jnp.concatenate([arr[:, k:], arr[:, :k]], axis=1)


def _stats_kernel(x_ref, out_ref):
    x = x_ref[...]
    s = jnp.sum(x, axis=(0, 2), keepdims=True)
    q = jnp.sum(x * x, axis=(0, 2), keepdims=True)
    out_ref[...] = jnp.concatenate([s, q], axis=2)


def _make_main_kernel(nb, C, C2, F, H, W):
    HW = H * W
    M = nb * HW
    offs = [(dy, dx) for dy in (-1, 0, 1) for dx in (-1, 0, 1)]

    def kern(x_ref, ms_ref, wc_ref, w0_ref, w2_ref, wz_ref, b01_ref, rs_ref,
             act_ref, w_out_ref, out_ref, det_ref):
        col = jax.lax.broadcasted_iota(jnp.int32, (1, M), 1)
        ml = col % HW
        xpos = ml % W
        ypos = ml // W
        oks = [((xpos + dx >= 0) & (xpos + dx < W) &
                (ypos + dy >= 0) & (ypos + dy < H)) for dy, dx in offs]

        if nb > 1:
            xs = jnp.concatenate([x_ref[i] for i in range(nb)], axis=1)
        else:
            xs = x_ref[0]

        a = ms_ref[:, 1:2] * (xs - ms_ref[:, 0:1])
        w = jnp.dot(wc_ref[...], a, preferred_element_type=jnp.float32)
        in_b = w[C2:C]

        base = w[0:8]
        slabs = [jnp.where(oks[t], _rot_lanes(base, dy * W + dx, M), 0.0)
                 for t, (dy, dx) in enumerate(offs)]
        patch = jnp.concatenate(slabs, axis=0).astype(jnp.bfloat16)
        h1 = jnp.dot(w0_ref[...], patch, preferred_element_type=jnp.float32)
        h1 = jnp.maximum(h1.astype(jnp.bfloat16) + b01_ref[:, 0:1], 0)

        h2 = jnp.dot(w2_ref[...], h1, preferred_element_type=jnp.float32)
        h2 = jnp.maximum(h2.astype(jnp.bfloat16) + b01_ref[:, 1:2], 0)

        G = jnp.dot(wz_ref[...], h2, preferred_element_type=jnp.float32)
        acc = None
        for t, (dy, dx) in enumerate(offs):
            g = G[16 * t:16 * t + C]
            contrib = jnp.where(oks[t], _rot_lanes(g, dy * W + dx, M),
                                rs_ref[:, t:t + 1])
            acc = contrib if acc is None else acc + contrib
        net = acc + rs_ref[:, 9:10]

        s = jax.nn.sigmoid(net[0:C2] + 2.0)
        out_b = (in_b + net[C2:C]) * s
        log_s = jnp.log(s)

        dets = []
        for i in range(nb):
            sl = slice(i * HW, (i + 1) * HW)
            act_ref[i] = a[:, sl]
            w_out_ref[i] = w[:, sl]
            out_ref[i] = jnp.concatenate([w[0:C2, sl], out_b[:, sl]], axis=0)
            dets.append(jnp.sum(log_s[:, sl]))

        rows = [jnp.zeros((1, 128), jnp.float32) + d for d in dets]
        if nb < 8:
            rows.append(jnp.zeros((8 - nb, 128), jnp.float32))
        det_ref[0] = jnp.concatenate(rows, axis=0)

    return kern


def kernel(w_p, w_l, w_u, s_sign, w_s, l_mask, u_mask,
           w0, b0, w2, b2, wz, bz, scale_z, x):
    N, C, H, W = x.shape
    C2 = C // 2
    F = w0.shape[0]
    HW = H * W
    M = N * HW
    x3 = x.reshape(N, C, HW)

    ns = _NS if N % _NS == 0 else N
    nb = _NB if N % _NB == 0 else 1
    G = N // ns

    parts = pl.pallas_call(
        _stats_kernel,
        grid=(G,),
        in_specs=[pl.BlockSpec((ns, C, HW), lambda g: (g, 0, 0))],
        out_specs=pl.BlockSpec((1, C, 2), lambda g: (g, 0, 0)),
        out_shape=jax.ShapeDtypeStruct((G, C, 2), jnp.float32),
        compiler_params=pltpu.CompilerParams(dimension_semantics=("parallel",)),
    )(x3)
    stats = jnp.sum(parts, axis=0)

    mean = stats[:, 0] / M
    var = (stats[:, 1] - M * mean * mean) / (M - 1)
    scale = 1.0 / (jnp.sqrt(var) + 1e-6)
    logdet_act = float(HW) * jnp.sum(jnp.log(jnp.abs(scale)))
    det1 = float(HW) * jnp.sum(w_s)
    ms = jnp.stack([mean, scale], axis=1)

    l = w_l * l_mask + jnp.eye(C, dtype=jnp.float32)
    u = w_u * u_mask + jnp.diag(s_sign * jnp.exp(w_s))
    wc = w_p @ l @ u

    w0_2d = w0.transpose(0, 2, 3, 1).reshape(F, 9 * C2)
    w0_al = jnp.zeros((F, 9 * 8), jnp.float32)
    w0_al = w0_al.reshape(F, 9, 8).at[:, :, 0:C2].set(
        w0_2d.reshape(F, 9, C2)).reshape(F, 72).astype(jnp.bfloat16)

    w2_2d = w2[:, :, 0, 0].astype(jnp.bfloat16)
    cs = jnp.exp(scale_z * 3.0)
    wz_t = (wz * cs[:, None, None, None]).transpose(2, 3, 0, 1).reshape(9, C, F)
    wz_st = jnp.zeros((9, 16, F), jnp.float32).at[:, 0:C, :].set(wz_t)
    wz_st = wz_st.reshape(144, F).astype(jnp.bfloat16)
    rs_mat = jnp.concatenate([jnp.sum(wz_t, axis=2).T, (bz * cs)[:, None]],
                             axis=1)
    b01 = jnp.stack([b0, b2], axis=1).astype(jnp.bfloat16)

    act3, w3, out3, det_blk = pl.pallas_call(
        _make_main_kernel(nb, C, C2, F, H, W),
        grid=(N // nb,),
        in_specs=[pl.BlockSpec((nb, C, HW), lambda n: (n, 0, 0)),
                  pl.BlockSpec((C, 2), lambda n: (0, 0)),
                  pl.BlockSpec((C, C), lambda n: (0, 0)),
                  pl.BlockSpec((F, 72), lambda n: (0, 0)),
                  pl.BlockSpec((F, F), lambda n: (0, 0)),
                  pl.BlockSpec((144, F), lambda n: (0, 0)),
                  pl.BlockSpec((F, 2), lambda n: (0, 0)),
                  pl.BlockSpec((C, 10), lambda n: (0, 0))],
        out_specs=[pl.BlockSpec((nb, C, HW), lambda n: (n, 0, 0)),
                   pl.BlockSpec((nb, C, HW), lambda n: (n, 0, 0)),
                   pl.BlockSpec((nb, C, HW), lambda n: (n, 0, 0)),
                   pl.BlockSpec((1, 8, 128), lambda n: (n, 0, 0))],
        out_shape=[jax.ShapeDtypeStruct((N, C, HW), jnp.float32),
                   jax.ShapeDtypeStruct((N, C, HW), jnp.float32),
                   jax.ShapeDtypeStruct((N, C, HW), jnp.float32),
                   jax.ShapeDtypeStruct((N // nb, 8, 128), jnp.float32)],
        compiler_params=pltpu.CompilerParams(dimension_semantics=("parallel",)),
    )(x3, ms, wc, w0_al, w2_2d, wz_st, b01, rs_mat)

    det2 = det_blk[:, 0:nb, 0].reshape(N)
    logdet = logdet_act + det1 + det2
    return (act3.reshape(N, C, H, W), w3.reshape(N, C, H, W),
            out3.reshape(N, C, H, W), logdet)


# speedup vs baseline: 4.9769x; 1.0393x over previous
import jax
import jax.numpy as jnp
from jax.experimental import pallas as pl
from jax.experimental.pallas import tpu as pltpu

_NB = 8
_NS = 64


def _rot_lanes(arr, k, size):
    k = k % size
    if k == 0:
        return arr
    return jnp.concatenate([arr[:, k:], arr[:, :k]], axis=1)


def _stats_kernel(x_ref, out_ref):
    x = x_ref[...]
    s = jnp.sum(x, axis=(0, 2), keepdims=True)
    q = jnp.sum(x * x, axis=(0, 2), keepdims=True)
    out_ref[...] = jnp.concatenate([s, q], axis=2)


def _make_main_kernel(nb, C, C2, F, H, W, N):
    HW = H * W
    M = nb * HW
    MT = float(N * HW)
    offs = [(dy, dx) for dy in (-1, 0, 1) for dx in (-1, 0, 1)]

    def kern(x_ref, st_ref, wp_ref, wl_ref, wu_ref, lm_ref, um_ref,
             ssn_ref, ws_ref, w0_ref, w2_ref, wz_ref, b01_ref, rs_ref,
             act_ref, w_out_ref, out_ref, det_ref):
        col = jax.lax.broadcasted_iota(jnp.int32, (1, M), 1)
        ml = col % HW
        xpos = ml % W
        ypos = ml // W
        oks = [((xpos + dx >= 0) & (xpos + dx < W) &
                (ypos + dy >= 0) & (ypos + dy < H)) for dy, dx in offs]

        mean = st_ref[:, 0:1] * (1.0 / MT)
        var = (st_ref[:, 1:2] - MT * mean * mean) * (1.0 / (MT - 1.0))
        scale = 1.0 / (jnp.sqrt(var) + 1e-6)
        logdet_const = float(HW) * (jnp.sum(jnp.log(scale))
                                    + jnp.sum(ws_ref[...]))

        ri = jax.lax.broadcasted_iota(jnp.int32, (C, C), 0)
        ci = jax.lax.broadcasted_iota(jnp.int32, (C, C), 1)
        eyem = ri == ci
        lmat = wl_ref[...] * lm_ref[...] + jnp.where(eyem, 1.0, 0.0)
        dvec = ssn_ref[...] * jnp.exp(ws_ref[...])
        umat = (wu_ref[...] * um_ref[...] +
                jnp.where(eyem, jnp.broadcast_to(dvec, (C, C)), 0.0))
        wc = jnp.dot(wp_ref[...],
                     jnp.dot(lmat, umat, preferred_element_type=jnp.float32),
                     preferred_element_type=jnp.float32)

        if nb > 1:
            xs = jnp.concatenate([x_ref[i] for i in range(nb)], axis=1)
        else:
            xs = x_ref[0]

        a = scale * (xs - mean)
        w = jnp.dot(wc, a, preferred_element_type=jnp.float32)
        in_b = w[C2:C]

        base = w[0:8].astype(jnp.bfloat16)
        zero_b = jnp.zeros((), jnp.bfloat16)
        slabs = [jnp.where(oks[t], _rot_lanes(base, dy * W + dx, M), zero_b)
                 for t, (dy, dx) in enumerate(offs)]
        patch = jnp.concatenate(slabs, axis=0)
        h1 = jnp.dot(w0_ref[...], patch, preferred_element_type=jnp.float32)
        h1 = jnp.maximum(h1.astype(jnp.bfloat16) + b01_ref[:, 0:1], 0)

        h2 = jnp.dot(w2_ref[...], h1, preferred_element_type=jnp.float32)
        h2 = jnp.maximum(h2.astype(jnp.bfloat16) + b01_ref[:, 1:2], 0)

        G = jnp.dot(wz_ref[...], h2, preferred_element_type=jnp.float32)
        acc = None
        for t, (dy, dx) in enumerate(offs):
            g = G[16 * t:16 * t + C]
            contrib = jnp.where(oks[t], _rot_lanes(g, dy * W + dx, M),
                                rs_ref[:, t:t + 1])
            acc = contrib if acc is None else acc + contrib
        net = acc + rs_ref[:, 9:10]

        s = jax.nn.sigmoid(net[0:C2] + 2.0)
        out_b = (in_b + net[C2:C]) * s
        log_s = jnp.log(s)

        dets = []
        for i in range(nb):
            sl = slice(i * HW, (i + 1) * HW)
            act_ref[i] = a[:, sl]
            w_out_ref[i] = w[:, sl]
            out_ref[i] = jnp.concatenate([w[0:C2, sl], out_b[:, sl]], axis=0)
            dets.append(jnp.sum(log_s[:, sl]) + logdet_const)

        rows = [jnp.zeros((1, 128), jnp.float32) + d for d in dets]
        if nb < 8:
            rows.append(jnp.zeros((8 - nb, 128), jnp.float32))
        det_ref[0] = jnp.concatenate(rows, axis=0)

    return kern


def kernel(w_p, w_l, w_u, s_sign, w_s, l_mask, u_mask,
           w0, b0, w2, b2, wz, bz, scale_z, x):
    N, C, H, W = x.shape
    C2 = C // 2
    F = w0.shape[0]
    HW = H * W
    M = N * HW
    x3 = x.reshape(N, C, HW)

    ns = _NS if N % _NS == 0 else N
    nb = _NB if N % _NB == 0 else 1
    G = N // ns

    parts = pl.pallas_call(
        _stats_kernel,
        grid=(G,),
        in_specs=[pl.BlockSpec((ns, C, HW), lambda g: (g, 0, 0))],
        out_specs=pl.BlockSpec((1, C, 2), lambda g: (g, 0, 0)),
        out_shape=jax.ShapeDtypeStruct((G, C, 2), jnp.float32),
        compiler_params=pltpu.CompilerParams(dimension_semantics=("parallel",)),
    )(x3)
    stats = jnp.sum(parts, axis=0)

    w0_al = jnp.pad(w0.transpose(0, 2, 3, 1), ((0, 0), (0, 0), (0, 0), (0, 2))
                    ).reshape(F, 72).astype(jnp.bfloat16)

    w2_2d = w2[:, :, 0, 0].astype(jnp.bfloat16)
    cs = jnp.exp(scale_z * 3.0)
    wzp = (wz * cs[:, None, None, None]).transpose(2, 3, 0, 1)
    wz_st = jnp.pad(wzp, ((0, 0), (0, 0), (0, 4), (0, 0))
                    ).reshape(144, F).astype(jnp.bfloat16)
    rs_mat = jnp.concatenate([jnp.sum(wzp, axis=3).reshape(9, C).T,
                              (bz * cs)[:, None]], axis=1)
    b01 = jnp.stack([b0, b2], axis=1).astype(jnp.bfloat16)

    cc_spec = pl.BlockSpec((C, C), lambda n: (0, 0))
    row_spec = pl.BlockSpec((1, C), lambda n: (0, 0))
    act3, w3, out3, det_blk = pl.pallas_call(
        _make_main_kernel(nb, C, C2, F, H, W, N),
        grid=(N // nb,),
        in_specs=[pl.BlockSpec((nb, C, HW), lambda n: (n, 0, 0)),
                  pl.BlockSpec((C, 2), lambda n: (0, 0)),
                  cc_spec, cc_spec, cc_spec, cc_spec, cc_spec,
                  row_spec, row_spec,
                  pl.BlockSpec((F, 72), lambda n: (0, 0)),
                  pl.BlockSpec((F, F), lambda n: (0, 0)),
                  pl.BlockSpec((144, F), lambda n: (0, 0)),
                  pl.BlockSpec((F, 2), lambda n: (0, 0)),
                  pl.BlockSpec((C, 10), lambda n: (0, 0))],
        out_specs=[pl.BlockSpec((nb, C, HW), lambda n: (n, 0, 0)),
                   pl.BlockSpec((nb, C, HW), lambda n: (n, 0, 0)),
                   pl.BlockSpec((nb, C, HW), lambda n: (n, 0, 0)),
                   pl.BlockSpec((1, 8, 128), lambda n: (n, 0, 0))],
        out_shape=[jax.ShapeDtypeStruct((N, C, HW), jnp.float32),
                   jax.ShapeDtypeStruct((N, C, HW), jnp.float32),
                   jax.ShapeDtypeStruct((N, C, HW), jnp.float32),
                   jax.ShapeDtypeStruct((N // nb, 8, 128), jnp.float32)],
        compiler_params=pltpu.CompilerParams(dimension_semantics=("parallel",)),
    )(x3, stats, w_p, w_l, w_u, l_mask, u_mask,
      s_sign.reshape(1, C), w_s.reshape(1, C),
      w0_al, w2_2d, wz_st, b01, rs_mat)

    logdet = det_blk[:, 0:nb, 0].reshape(N)
    return (act3.reshape(N, C, H, W), w3.reshape(N, C, H, W),
            out3.reshape(N, C, H, W), logdet)
